# baseline (speedup 1.0000x reference)
import jax
import jax.numpy as jnp
from jax import lax
from jax.experimental import pallas as pl
from jax.experimental.pallas import tpu as pltpu
from jax.experimental.pallas import tpu_sc as plsc

N = 10000
NP = 10240
E = 320000
D = 128
NC, NS = 2, 16
NW = NC * NS
CHUNK = 128
NBLK = E // CHUNK
BIG = 79
SMALL = 78
WIN = 20
NFULL = 3
ROWS_W = NP // NS
ZCH = 128
DEG_W = 16

_mesh = plsc.VectorSubcoreMesh(
    core_axis_name="c", subcore_axis_name="s", num_cores=NC, num_subcores=NS)

_NOTILE = pltpu.CompilerParams(use_tc_tiling_on_sc=False)
if "needs_layout_passes" in pltpu.CompilerParams.__dataclass_fields__:
    import dataclasses as _dc
    _NOTILE = _dc.replace(_NOTILE, needs_layout_passes=False)


def _tile_range(wid):
    start = SMALL * wid + jnp.minimum(wid, NBLK - SMALL * NW)
    is_big = wid < NBLK - SMALL * NW
    return start, is_big


def _deg_body(e3_hbm, deg_hbm, ev, onesv, zerov, deg_sh):
    ci = lax.axis_index("c")
    si = lax.axis_index("s")
    wid = ci * NS + si
    start, is_big = _tile_range(wid)

    @pl.when(is_big)
    def _():
        pltpu.sync_copy(e3_hbm.at[pl.ds(start, BIG)], ev)

    @pl.when(jnp.logical_not(is_big))
    def _():
        pltpu.sync_copy(e3_hbm.at[pl.ds(start, SMALL)], ev.at[pl.ds(0, SMALL)])

    @pl.loop(0, CHUNK)
    def _(r):
        onesv[r, pl.ds(0, DEG_W)] = jnp.ones((DEG_W,), jnp.float32)

    @pl.loop(0, ZCH)
    def _(r):
        zerov[r, pl.ds(0, DEG_W)] = jnp.zeros((DEG_W,), jnp.float32)

    @pl.loop(0, ROWS_W // ZCH)
    def _(k):
        pltpu.sync_copy(zerov, deg_sh.at[pl.ds(si * ROWS_W + k * ZCH, ZCH)])

    plsc.subcore_barrier()

    @pl.loop(0, SMALL)
    def _(j):
        pltpu.sync_copy(onesv, deg_sh.at[ev.at[j, 1]], add=True)

    @pl.when(is_big)
    def _():
        pltpu.sync_copy(onesv, deg_sh.at[ev.at[SMALL, 1]], add=True)

    plsc.subcore_barrier()

    sl = pl.ds(si * ROWS_W, ROWS_W)
    pltpu.sync_copy(deg_sh.at[sl], deg_hbm.at[ci].at[sl])


def _agg_body(e3_hbm, gp_hbm, out_hbm, ev, pb, fb0, fb1, acc,
              sem0, ssem0, ssem1):
    ci = lax.axis_index("c")
    si = lax.axis_index("s")
    wid = ci * NS + si
    start, is_big = _tile_range(wid)

    @pl.loop(0, ZCH)
    def _(r):
        @pl.loop(0, D // 16)
        def _(q):
            fb0[r, pl.ds(q * 16, 16)] = jnp.zeros((16,), jnp.float32)

    @pl.loop(0, ROWS_W // ZCH)
    def _(k):
        pltpu.sync_copy(fb0, acc.at[pl.ds(si * ROWS_W + k * ZCH, ZCH)])

    plsc.subcore_barrier()

    def convert(fb):
        @pl.loop(0, CHUNK)
        def _(r):
            for t in range(D // 32):
                v = pb[r, pl.ds(t * 16, 16)]
                lo = plsc.bitcast(v << 16, jnp.float32)
                hi = plsc.bitcast(v & jnp.int32(-65536), jnp.float32)
                fb[r, pl.ds(32 * t, 16)] = lo
                fb[r, pl.ds(32 * t + 16, 16)] = hi

    def emit(a, b):
        npair = (b - a) // 2
        pltpu.async_copy(gp_hbm.at[ev.at[a, 0]], pb, sem0)

        @pl.loop(0, npair)
        def _(p):
            j0 = a + 2 * p
            j1 = j0 + 1
            pltpu.make_async_copy(gp_hbm.at[ev.at[j0, 0]], pb, sem0).wait()

            @pl.when(p > 0)
            def _():
                pltpu.make_async_copy(fb0, acc.at[ev.at[j0 - 2, 1]],
                                      ssem0).wait()

            convert(fb0)
            pltpu.async_copy(gp_hbm.at[ev.at[j1, 0]], pb, sem0)
            pltpu.async_copy(fb0, acc.at[ev.at[j0, 1]], ssem0, add=True)

            pltpu.make_async_copy(gp_hbm.at[ev.at[j1, 0]], pb, sem0).wait()

            @pl.when(p > 0)
            def _():
                pltpu.make_async_copy(fb1, acc.at[ev.at[j1 - 2, 1]],
                                      ssem1).wait()

            convert(fb1)

            @pl.when(p < npair - 1)
            def _():
                pltpu.async_copy(gp_hbm.at[ev.at[j1 + 1, 0]], pb, sem0)

            pltpu.async_copy(fb1, acc.at[ev.at[j1, 1]], ssem1, add=True)

        pltpu.make_async_copy(fb0, acc.at[ev.at[b - 2, 1]], ssem0).wait()
        pltpu.make_async_copy(fb1, acc.at[ev.at[b - 1, 1]], ssem1).wait()

    def solo(j):
        pltpu.async_copy(gp_hbm.at[ev.at[j, 0]], pb, sem0).wait()
        convert(fb0)
        pltpu.sync_copy(fb0, acc.at[ev.at[j, 1]], add=True)

    for w in range(NFULL):
        pltpu.sync_copy(e3_hbm.at[pl.ds(start + WIN * w, WIN)], ev)
        emit(0, WIN)

    @pl.when(is_big)
    def _():
        pltpu.sync_copy(e3_hbm.at[pl.ds(start + BIG - WIN, WIN)], ev)
        solo(1)
        emit(2, WIN)

    @pl.when(jnp.logical_not(is_big))
    def _():
        pltpu.sync_copy(e3_hbm.at[pl.ds(start + SMALL - WIN, WIN)], ev)
        emit(2, WIN)

    plsc.subcore_barrier()

    sl = pl.ds(si * ROWS_W, ROWS_W)
    pltpu.sync_copy(acc.at[sl], out_hbm.at[ci].at[sl])


def _make_sc_kernels(interpret=False):
    deg_k = pl.kernel(
        _deg_body,
        out_type=jax.ShapeDtypeStruct((NC, NP, DEG_W), jnp.float32),
        mesh=_mesh,
        scratch_types=[
            pltpu.VMEM((BIG, 2, CHUNK), jnp.int32),
            pltpu.VMEM((CHUNK, DEG_W), jnp.float32),
            pltpu.VMEM((ZCH, DEG_W), jnp.float32),
            pltpu.VMEM_SHARED((NP, DEG_W), jnp.float32),
        ],
        compiler_params=_NOTILE,
        interpret=interpret,
    )
    agg_k = pl.kernel(
        _agg_body,
        out_type=jax.ShapeDtypeStruct((NC, NP, D), jnp.float32),
        mesh=_mesh,
        scratch_types=[
            pltpu.VMEM((WIN, 2, CHUNK), jnp.int32),
            pltpu.VMEM((CHUNK, D // 2), jnp.int32),
            pltpu.VMEM((ZCH, D), jnp.float32),
            pltpu.VMEM((ZCH, D), jnp.float32),
            pltpu.VMEM_SHARED((NP, D), jnp.float32),
            pltpu.SemaphoreType.DMA,
            pltpu.SemaphoreType.DMA,
            pltpu.SemaphoreType.DMA,
        ],
        compiler_params=_NOTILE,
        interpret=interpret,
    )
    return deg_k, agg_k


_deg_kernel, _agg_kernel = _make_sc_kernels()


def _dinv_block(deg_ref):
    deg = deg_ref[0, :, 0:1] + deg_ref[1, :, 0:1]
    return jnp.where(deg > 0.0, lax.rsqrt(jnp.maximum(deg, 1.0)), 0.0)


def _scale_body(deg_ref, x_ref, w_ref, gp_ref):
    h = jnp.dot(x_ref[...], w_ref[...],
                preferred_element_type=jnp.float32)
    g = _dinv_block(deg_ref) * h
    g4 = g.reshape(BLK, D // 32, 32)
    au = lax.bitcast_convert_type(
        g4[:, :, 0:16].astype(jnp.bfloat16), jnp.uint16).astype(jnp.uint32)
    bu = lax.bitcast_convert_type(
        g4[:, :, 16:32].astype(jnp.bfloat16), jnp.uint16).astype(jnp.uint32)
    packed = lax.bitcast_convert_type(au | (bu << 16), jnp.int32)
    gp_ref[...] = packed.reshape(BLK, D // 2)


def _out_body(deg_ref, acc_ref, b_ref, o_ref):
    s = acc_ref[0] + acc_ref[1]
    o_ref[...] = jnp.maximum(_dinv_block(deg_ref) * s + b_ref[...], 0.0)


BLK = 1000


def _scale_call(deg, x, W, interpret=False):
    return pl.pallas_call(
        _scale_body,
        grid=(N // BLK,),
        in_specs=[
            pl.BlockSpec((NC, BLK, DEG_W), lambda i: (0, i, 0)),
            pl.BlockSpec((BLK, D), lambda i: (i, 0)),
            pl.BlockSpec((D, D), lambda i: (0, 0)),
        ],
        out_specs=pl.BlockSpec((BLK, D // 2), lambda i: (i, 0)),
        out_shape=jax.ShapeDtypeStruct((N, D // 2), jnp.int32),
        interpret=interpret,
    )(deg, x, W)


def _out_call(deg, acc, b2, interpret=False):
    return pl.pallas_call(
        _out_body,
        grid=(N // BLK,),
        in_specs=[
            pl.BlockSpec((NC, BLK, DEG_W), lambda i: (0, i, 0)),
            pl.BlockSpec((NC, BLK, D), lambda i: (0, i, 0)),
            pl.BlockSpec((1, D), lambda i: (0, 0)),
        ],
        out_specs=pl.BlockSpec((BLK, D), lambda i: (i, 0)),
        out_shape=jax.ShapeDtypeStruct((N, D), jnp.float32),
        interpret=interpret,
    )(deg, acc, b2)


def kernel(x, edge_index, W, b):
    e3 = jnp.transpose(
        edge_index.astype(jnp.int32).reshape(2, NBLK, CHUNK), (1, 0, 2))

    deg = _deg_kernel(e3)
    g = _scale_call(deg, x, W)
    acc = _agg_kernel(e3, g)
    return _out_call(deg, acc, b.reshape(1, D))


# speedup vs baseline: 1.2245x; 1.2245x over previous
import jax
import jax.numpy as jnp
from jax import lax
from jax.experimental import pallas as pl
from jax.experimental.pallas import tpu as pltpu
from jax.experimental.pallas import tpu_sc as plsc

N = 10000
NP = 10240
E = 320000
D = 128
NC, NS = 2, 16
NW = NC * NS
CHUNK = 128
NBLK = E // CHUNK
BIG = 79
SMALL = 78
WIN = 20
NFULL = 3
ROWS_W = NP // NS
ZCH = 128
DEG_W = 16

_mesh = plsc.VectorSubcoreMesh(
    core_axis_name="c", subcore_axis_name="s", num_cores=NC, num_subcores=NS)

_NOTILE = pltpu.CompilerParams(use_tc_tiling_on_sc=False)
if "needs_layout_passes" in pltpu.CompilerParams.__dataclass_fields__:
    import dataclasses as _dc
    _NOTILE = _dc.replace(_NOTILE, needs_layout_passes=False)


def _tile_range(wid):
    start = SMALL * wid + jnp.minimum(wid, NBLK - SMALL * NW)
    is_big = wid < NBLK - SMALL * NW
    return start, is_big


def _deg_body(e3_hbm, deg_hbm, ev, onesv, zerov, deg_sh):
    ci = lax.axis_index("c")
    si = lax.axis_index("s")
    wid = ci * NS + si
    start, is_big = _tile_range(wid)

    @pl.when(is_big)
    def _():
        pltpu.sync_copy(e3_hbm.at[pl.ds(start, BIG)], ev)

    @pl.when(jnp.logical_not(is_big))
    def _():
        pltpu.sync_copy(e3_hbm.at[pl.ds(start, SMALL)], ev.at[pl.ds(0, SMALL)])

    @pl.loop(0, CHUNK)
    def _(r):
        onesv[r, pl.ds(0, DEG_W)] = jnp.ones((DEG_W,), jnp.float32)

    @pl.loop(0, ZCH)
    def _(r):
        zerov[r, pl.ds(0, DEG_W)] = jnp.zeros((DEG_W,), jnp.float32)

    @pl.loop(0, ROWS_W // ZCH)
    def _(k):
        pltpu.sync_copy(zerov, deg_sh.at[pl.ds(si * ROWS_W + k * ZCH, ZCH)])

    plsc.subcore_barrier()

    @pl.loop(0, SMALL)
    def _(j):
        pltpu.sync_copy(onesv, deg_sh.at[ev.at[j, 1]], add=True)

    @pl.when(is_big)
    def _():
        pltpu.sync_copy(onesv, deg_sh.at[ev.at[SMALL, 1]], add=True)

    plsc.subcore_barrier()

    sl = pl.ds(si * ROWS_W, ROWS_W)
    pltpu.sync_copy(deg_sh.at[sl], deg_hbm.at[ci].at[sl])


HC = CHUNK // 2


def _agg_body(e4_hbm, gp_hbm, out_hbm, ev, pb0, pb1, fb0, fb1, acc,
              sem0, sem1, ssem0, ssem1):
    ci = lax.axis_index("c")
    si = lax.axis_index("s")
    wid = ci * NS + si
    start, is_big = _tile_range(wid)

    @pl.loop(0, HC)
    def _(r):
        @pl.loop(0, D // 16)
        def _(q):
            fb0[r, pl.ds(q * 16, 16)] = jnp.zeros((16,), jnp.float32)

    @pl.loop(0, ROWS_W // HC)
    def _(k):
        pltpu.sync_copy(fb0, acc.at[pl.ds(si * ROWS_W + k * HC, HC)])

    plsc.subcore_barrier()

    def convert(pb, fb):
        @pl.loop(0, HC // 4)
        def _(r4):
            for u in range(4):
                r = r4 * 4 + u
                for t in range(D // 32):
                    v = pb[r, pl.ds(t * 16, 16)]
                    fb[r, pl.ds(32 * t, 16)] = plsc.bitcast(
                        v << 16, jnp.float32)
                    fb[r, pl.ds(32 * t + 16, 16)] = plsc.bitcast(
                        v & jnp.int32(-65536), jnp.float32)

    def emit(a, b):
        pltpu.async_copy(gp_hbm.at[ev.at[a, 0, 0]], pb0, sem0)
        pltpu.async_copy(gp_hbm.at[ev.at[a, 0, 1]], pb1, sem1)

        @pl.loop(a, b)
        def _(j):
            pltpu.make_async_copy(gp_hbm.at[ev.at[j, 0, 0]], pb0, sem0).wait()

            @pl.when(j > a)
            def _():
                pltpu.make_async_copy(fb0, acc.at[ev.at[j - 1, 1, 0]],
                                      ssem0).wait()

            convert(pb0, fb0)

            @pl.when(j < b - 1)
            def _():
                pltpu.async_copy(gp_hbm.at[ev.at[j + 1, 0, 0]], pb0, sem0)

            pltpu.async_copy(fb0, acc.at[ev.at[j, 1, 0]], ssem0, add=True)

            pltpu.make_async_copy(gp_hbm.at[ev.at[j, 0, 1]], pb1, sem1).wait()

            @pl.when(j > a)
            def _():
                pltpu.make_async_copy(fb1, acc.at[ev.at[j - 1, 1, 1]],
                                      ssem1).wait()

            convert(pb1, fb1)

            @pl.when(j < b - 1)
            def _():
                pltpu.async_copy(gp_hbm.at[ev.at[j + 1, 0, 1]], pb1, sem1)

            pltpu.async_copy(fb1, acc.at[ev.at[j, 1, 1]], ssem1, add=True)

        pltpu.make_async_copy(fb0, acc.at[ev.at[b - 1, 1, 0]], ssem0).wait()
        pltpu.make_async_copy(fb1, acc.at[ev.at[b - 1, 1, 1]], ssem1).wait()

    def solo(j):
        for h in range(2):
            pltpu.async_copy(gp_hbm.at[ev.at[j, 0, h]], pb0, sem0).wait()
            convert(pb0, fb0)
            pltpu.sync_copy(fb0, acc.at[ev.at[j, 1, h]], add=True)

    for w in range(NFULL):
        pltpu.sync_copy(e4_hbm.at[pl.ds(start + WIN * w, WIN)], ev)
        emit(0, WIN)

    @pl.when(is_big)
    def _():
        pltpu.sync_copy(e4_hbm.at[pl.ds(start + BIG - WIN, WIN)], ev)
        solo(1)
        emit(2, WIN)

    @pl.when(jnp.logical_not(is_big))
    def _():
        pltpu.sync_copy(e4_hbm.at[pl.ds(start + SMALL - WIN, WIN)], ev)
        emit(2, WIN)

    plsc.subcore_barrier()

    sl = pl.ds(si * ROWS_W, ROWS_W)
    pltpu.sync_copy(acc.at[sl], out_hbm.at[ci].at[sl])


def _make_sc_kernels(interpret=False):
    deg_k = pl.kernel(
        _deg_body,
        out_type=jax.ShapeDtypeStruct((NC, NP, DEG_W), jnp.float32),
        mesh=_mesh,
        scratch_types=[
            pltpu.VMEM((BIG, 2, CHUNK), jnp.int32),
            pltpu.VMEM((CHUNK, DEG_W), jnp.float32),
            pltpu.VMEM((ZCH, DEG_W), jnp.float32),
            pltpu.VMEM_SHARED((NP, DEG_W), jnp.float32),
        ],
        compiler_params=_NOTILE,
        interpret=interpret,
    )
    agg_k = pl.kernel(
        _agg_body,
        out_type=jax.ShapeDtypeStruct((NC, NP, D), jnp.float32),
        mesh=_mesh,
        scratch_types=[
            pltpu.VMEM((WIN, 2, 2, HC), jnp.int32),
            pltpu.VMEM((HC, D // 2), jnp.int32),
            pltpu.VMEM((HC, D // 2), jnp.int32),
            pltpu.VMEM((HC, D), jnp.float32),
            pltpu.VMEM((HC, D), jnp.float32),
            pltpu.VMEM_SHARED((NP, D), jnp.float32),
            pltpu.SemaphoreType.DMA,
            pltpu.SemaphoreType.DMA,
            pltpu.SemaphoreType.DMA,
            pltpu.SemaphoreType.DMA,
        ],
        compiler_params=_NOTILE,
        interpret=interpret,
    )
    return deg_k, agg_k


_deg_kernel, _agg_kernel = _make_sc_kernels()


def _dinv_block(deg_ref):
    deg = deg_ref[0, :, 0:1] + deg_ref[1, :, 0:1]
    return jnp.where(deg > 0.0, lax.rsqrt(jnp.maximum(deg, 1.0)), 0.0)


def _scale_body(deg_ref, x_ref, w_ref, gp_ref):
    h = jnp.dot(x_ref[...], w_ref[...],
                preferred_element_type=jnp.float32)
    g = _dinv_block(deg_ref) * h
    g4 = g.reshape(BLK, D // 32, 32)
    au = lax.bitcast_convert_type(
        g4[:, :, 0:16].astype(jnp.bfloat16), jnp.uint16).astype(jnp.uint32)
    bu = lax.bitcast_convert_type(
        g4[:, :, 16:32].astype(jnp.bfloat16), jnp.uint16).astype(jnp.uint32)
    packed = lax.bitcast_convert_type(au | (bu << 16), jnp.int32)
    gp_ref[...] = packed.reshape(BLK, D // 2)


def _out_body(deg_ref, acc_ref, b_ref, o_ref):
    s = acc_ref[0] + acc_ref[1]
    o_ref[...] = jnp.maximum(_dinv_block(deg_ref) * s + b_ref[...], 0.0)


BLK = 1000


def _scale_call(deg, x, W, interpret=False):
    return pl.pallas_call(
        _scale_body,
        grid=(N // BLK,),
        in_specs=[
            pl.BlockSpec((NC, BLK, DEG_W), lambda i: (0, i, 0)),
            pl.BlockSpec((BLK, D), lambda i: (i, 0)),
            pl.BlockSpec((D, D), lambda i: (0, 0)),
        ],
        out_specs=pl.BlockSpec((BLK, D // 2), lambda i: (i, 0)),
        out_shape=jax.ShapeDtypeStruct((N, D // 2), jnp.int32),
        interpret=interpret,
    )(deg, x, W)


def _out_call(deg, acc, b2, interpret=False):
    return pl.pallas_call(
        _out_body,
        grid=(N // BLK,),
        in_specs=[
            pl.BlockSpec((NC, BLK, DEG_W), lambda i: (0, i, 0)),
            pl.BlockSpec((NC, BLK, D), lambda i: (0, i, 0)),
            pl.BlockSpec((1, D), lambda i: (0, 0)),
        ],
        out_specs=pl.BlockSpec((BLK, D), lambda i: (i, 0)),
        out_shape=jax.ShapeDtypeStruct((N, D), jnp.float32),
        interpret=interpret,
    )(deg, acc, b2)


def kernel(x, edge_index, W, b):
    e3 = jnp.transpose(
        edge_index.astype(jnp.int32).reshape(2, NBLK, CHUNK), (1, 0, 2))
    e4 = e3.reshape(NBLK, 2, 2, HC)

    deg = _deg_kernel(e3)
    gp = _scale_call(deg, x, W)
    acc = _agg_kernel(e4, gp)
    return _out_call(deg, acc, b.reshape(1, D))


# speedup vs baseline: 1.7547x; 1.4330x over previous
import jax
import jax.numpy as jnp
from jax import lax
from jax.experimental import pallas as pl
from jax.experimental.pallas import tpu as pltpu
from jax.experimental.pallas import tpu_sc as plsc

N = 10000
NP = 10240
E = 320000
D = 128
NC, NS = 2, 16
NW = NC * NS
CHUNK = 128
NBLK = E // CHUNK
BIG = 79
SMALL = 78
WIN = 40
ROWS_W = NP // NS
ZCH = 128
DEG_W = 16

_mesh = plsc.VectorSubcoreMesh(
    core_axis_name="c", subcore_axis_name="s", num_cores=NC, num_subcores=NS)

_NOTILE = pltpu.CompilerParams(use_tc_tiling_on_sc=False)


def _tile_range(wid):
    start = SMALL * wid + jnp.minimum(wid, NBLK - SMALL * NW)
    is_big = wid < NBLK - SMALL * NW
    return start, is_big


def _deg_body(e3_hbm, deg_hbm, ev, onesv, zerov, deg_sh):
    ci = lax.axis_index("c")
    si = lax.axis_index("s")
    wid = ci * NS + si
    start, is_big = _tile_range(wid)

    @pl.when(is_big)
    def _():
        pltpu.sync_copy(e3_hbm.at[pl.ds(start, BIG)], ev)

    @pl.when(jnp.logical_not(is_big))
    def _():
        pltpu.sync_copy(e3_hbm.at[pl.ds(start, SMALL)], ev.at[pl.ds(0, SMALL)])

    @pl.loop(0, CHUNK)
    def _(r):
        onesv[r, pl.ds(0, DEG_W)] = jnp.ones((DEG_W,), jnp.float32)

    @pl.loop(0, ZCH)
    def _(r):
        zerov[r, pl.ds(0, DEG_W)] = jnp.zeros((DEG_W,), jnp.float32)

    @pl.loop(0, ROWS_W // ZCH)
    def _(k):
        pltpu.sync_copy(zerov, deg_sh.at[pl.ds(si * ROWS_W + k * ZCH, ZCH)])

    plsc.subcore_barrier()

    @pl.loop(0, SMALL)
    def _(j):
        pltpu.sync_copy(onesv, deg_sh.at[ev.at[j, 1]], add=True)

    @pl.when(is_big)
    def _():
        pltpu.sync_copy(onesv, deg_sh.at[ev.at[SMALL, 1]], add=True)

    plsc.subcore_barrier()

    sl = pl.ds(si * ROWS_W, ROWS_W)
    pltpu.sync_copy(deg_sh.at[sl], deg_hbm.at[ci].at[sl])


def _agg_body(e3_hbm, g_hbm, out_hbm, ev, buf0, buf1, acc,
              sem0, sem1, ssem0, ssem1):
    ci = lax.axis_index("c")
    si = lax.axis_index("s")
    wid = ci * NS + si
    start, is_big = _tile_range(wid)

    @pl.loop(0, ZCH)
    def _(r):
        @pl.loop(0, D // 16)
        def _(q):
            buf0[r, pl.ds(q * 16, 16)] = jnp.zeros((16,), jnp.float32)

    @pl.loop(0, ROWS_W // ZCH)
    def _(k):
        pltpu.sync_copy(buf0, acc.at[pl.ds(si * ROWS_W + k * ZCH, ZCH)])

    plsc.subcore_barrier()

    def solo(j):
        pltpu.async_copy(g_hbm.at[ev.at[j, 0]], buf0, sem0).wait()
        pltpu.sync_copy(buf0, acc.at[ev.at[j, 1]], add=True)

    def pairs(a, b):
        npair = (b - a) // 2
        pltpu.async_copy(g_hbm.at[ev.at[a, 0]], buf0, sem0)
        pltpu.async_copy(g_hbm.at[ev.at[a + 1, 0]], buf1, sem1)

        @pl.loop(0, npair)
        def _(p):
            e0 = a + 2 * p
            e1 = e0 + 1
            pltpu.make_async_copy(g_hbm.at[ev.at[e0, 0]], buf0, sem0).wait()
            pltpu.async_copy(buf0, acc.at[ev.at[e0, 1]], ssem0, add=True)
            pltpu.make_async_copy(g_hbm.at[ev.at[e1, 0]], buf1, sem1).wait()
            pltpu.async_copy(buf1, acc.at[ev.at[e1, 1]], ssem1, add=True)

            @pl.when(p < npair - 1)
            def _():
                pltpu.make_async_copy(buf0, acc.at[ev.at[e0, 1]], ssem0).wait()
                pltpu.async_copy(g_hbm.at[ev.at[e0 + 2, 0]], buf0, sem0)
                pltpu.make_async_copy(buf1, acc.at[ev.at[e1, 1]], ssem1).wait()
                pltpu.async_copy(g_hbm.at[ev.at[e1 + 2, 0]], buf1, sem1)

        pltpu.make_async_copy(buf0, acc.at[ev.at[b - 2, 1]], ssem0).wait()
        pltpu.make_async_copy(buf1, acc.at[ev.at[b - 1, 1]], ssem1).wait()

    pltpu.sync_copy(e3_hbm.at[pl.ds(start, WIN)], ev)
    pairs(0, WIN)

    @pl.when(is_big)
    def _():
        pltpu.sync_copy(e3_hbm.at[pl.ds(start + BIG - WIN, WIN)], ev)
        solo(1)
        pairs(2, WIN)

    @pl.when(jnp.logical_not(is_big))
    def _():
        pltpu.sync_copy(e3_hbm.at[pl.ds(start + SMALL - WIN, WIN)], ev)
        pairs(2, WIN)

    plsc.subcore_barrier()

    sl = pl.ds(si * ROWS_W, ROWS_W)
    pltpu.sync_copy(acc.at[sl], out_hbm.at[ci].at[sl])


def _make_sc_kernels(interpret=False):
    deg_k = pl.kernel(
        _deg_body,
        out_type=jax.ShapeDtypeStruct((NC, NP, DEG_W), jnp.float32),
        mesh=_mesh,
        scratch_types=[
            pltpu.VMEM((BIG, 2, CHUNK), jnp.int32),
            pltpu.VMEM((CHUNK, DEG_W), jnp.float32),
            pltpu.VMEM((ZCH, DEG_W), jnp.float32),
            pltpu.VMEM_SHARED((NP, DEG_W), jnp.float32),
        ],
        compiler_params=_NOTILE,
        interpret=interpret,
    )
    agg_k = pl.kernel(
        _agg_body,
        out_type=jax.ShapeDtypeStruct((NC, NP, D), jnp.float32),
        mesh=_mesh,
        scratch_types=[
            pltpu.VMEM((WIN, 2, CHUNK), jnp.int32),
            pltpu.VMEM((ZCH, D), jnp.float32),
            pltpu.VMEM((ZCH, D), jnp.float32),
            pltpu.VMEM_SHARED((NP, D), jnp.float32),
            pltpu.SemaphoreType.DMA,
            pltpu.SemaphoreType.DMA,
            pltpu.SemaphoreType.DMA,
            pltpu.SemaphoreType.DMA,
        ],
        compiler_params=_NOTILE,
        interpret=interpret,
    )
    return deg_k, agg_k


_deg_kernel, _agg_kernel = _make_sc_kernels()


def _dinv_block(deg_ref):
    deg = deg_ref[0, :, 0:1] + deg_ref[1, :, 0:1]
    return jnp.where(deg > 0.0, lax.rsqrt(jnp.maximum(deg, 1.0)), 0.0)


def _scale_body(deg_ref, x_ref, w_ref, g_ref):
    h = jnp.dot(x_ref[...], w_ref[...],
                preferred_element_type=jnp.float32)
    g_ref[...] = _dinv_block(deg_ref) * h


def _out_body(deg_ref, acc_ref, b_ref, o_ref):
    s = acc_ref[0] + acc_ref[1]
    o_ref[...] = jnp.maximum(_dinv_block(deg_ref) * s + b_ref[...], 0.0)


BLK = 1000


def _scale_call(deg, x, W, interpret=False):
    return pl.pallas_call(
        _scale_body,
        grid=(N // BLK,),
        in_specs=[
            pl.BlockSpec((NC, BLK, DEG_W), lambda i: (0, i, 0)),
            pl.BlockSpec((BLK, D), lambda i: (i, 0)),
            pl.BlockSpec((D, D), lambda i: (0, 0)),
        ],
        out_specs=pl.BlockSpec((BLK, D), lambda i: (i, 0)),
        out_shape=jax.ShapeDtypeStruct((N, D), jnp.float32),
        interpret=interpret,
    )(deg, x, W)


def _out_call(deg, acc, b2, interpret=False):
    return pl.pallas_call(
        _out_body,
        grid=(N // BLK,),
        in_specs=[
            pl.BlockSpec((NC, BLK, DEG_W), lambda i: (0, i, 0)),
            pl.BlockSpec((NC, BLK, D), lambda i: (0, i, 0)),
            pl.BlockSpec((1, D), lambda i: (0, 0)),
        ],
        out_specs=pl.BlockSpec((BLK, D), lambda i: (i, 0)),
        out_shape=jax.ShapeDtypeStruct((N, D), jnp.float32),
        interpret=interpret,
    )(deg, acc, b2)


def kernel(x, edge_index, W, b):
    e3 = jnp.transpose(
        edge_index.astype(jnp.int32).reshape(2, NBLK, CHUNK), (1, 0, 2))

    deg = _deg_kernel(e3)
    g = _scale_call(deg, x, W)
    acc = _agg_kernel(e3, g)
    return _out_call(deg, acc, b.reshape(1, D))
